# Initial kernel scaffold; baseline (speedup 1.0000x reference)
import jax
import jax.numpy as jnp
from jax.experimental import pallas as pl


def kernel(tag1, tag2, tag3, tag4, tag5, ind1, ind2, ind3, ind4, ind5, mask):
    raise NotImplementedError("write your pallas kernel here")


# speedup vs baseline: 1.2229x; 1.2229x over previous
import functools

import jax
import jax.numpy as jnp
from jax import lax
from jax.experimental import pallas as pl
from jax.experimental.pallas import tpu as pltpu
from jax.experimental.pallas import tpu_sc as plsc

_B, _C, _H, _W, _N = 64, 1, 256, 256, 256
_HW = _H * _W
_NC, _NS, _L = 2, 16, 16
_NW = _NC * _NS
_RPW = _B // _NW
_NCH = _N // 128
_NT = 5


def _pull_body(t1, t2, t3, t4, t5, ind_hbm, mask_hbm, out_hbm,
               idx_v, g_v, m_v, acc_v, sem):
    tags = (t1, t2, t3, t4, t5)
    wid = lax.axis_index("s") * _NC + lax.axis_index("c")

    acc = jnp.zeros((_L,), jnp.float32)
    for rb in range(_RPW):
        b = wid * _RPW + rb
        pltpu.sync_copy(mask_hbm.at[b], m_v)
        for t in range(_NT):
            pltpu.sync_copy(ind_hbm.at[t * _B + b], idx_v.at[t])
        copies = []
        for t in range(_NT):
            for c in range(_NCH):
                copies.append(pltpu.async_copy(
                    tags[t].at[idx_v.at[t].at[c]], g_v.at[t].at[c], sem))
        for cp in copies:
            cp.wait()

        num_vec = jnp.zeros((_L,), jnp.int32)
        for c in range(_NCH):
            for k in range(128 // _L):
                mb = m_v[c, pl.ds(k * _L, _L)] > 0
                num_vec = num_vec + plsc.all_reduce_population_count(mb)
        scale = 1.0 / (num_vec.astype(jnp.float32) + 1e-4)

        for c in range(_NCH):
            for k in range(128 // _L):
                sl = pl.ds(k * _L, _L)
                g = [g_v[t, c, sl] for t in range(_NT)]
                tm = (g[0] + g[1] + g[2] + g[3] + g[4]) * 0.2
                ssd = jnp.zeros((_L,), jnp.float32)
                for t in range(_NT):
                    d = g[t] - tm
                    ssd = ssd + d * d
                mf = m_v[c, sl].astype(jnp.float32)
                acc = acc + (mf * ssd) * scale

    acc_v[...] = acc
    pltpu.sync_copy(acc_v, out_hbm.at[wid])


@jax.jit
def _ae_pull(t1, t2, t3, t4, t5, ind_all, mask_i32):
    mesh = plsc.VectorSubcoreMesh(core_axis_name="c", subcore_axis_name="s")
    run = functools.partial(
        pl.kernel,
        mesh=mesh,
        compiler_params=pltpu.CompilerParams(needs_layout_passes=False),
        out_type=jax.ShapeDtypeStruct((_NW, _L), jnp.float32),
        scratch_types=[
            pltpu.VMEM((_NT, _NCH, 128), jnp.int32),
            pltpu.VMEM((_NT, _NCH, 128), jnp.float32),
            pltpu.VMEM((_NCH, 128), jnp.int32),
            pltpu.VMEM((_L,), jnp.float32),
            pltpu.SemaphoreType.DMA,
        ],
    )(_pull_body)
    return run(t1, t2, t3, t4, t5, ind_all, mask_i32)


def kernel(tag1, tag2, tag3, tag4, tag5, ind1, ind2, ind3, ind4, ind5, mask):
    flats = [t.reshape(_B * _HW) for t in (tag1, tag2, tag3, tag4, tag5)]
    offs = (jnp.arange(_B, dtype=jnp.int32) * _HW)[:, None]
    ind_all = jnp.stack([i + offs for i in (ind1, ind2, ind3, ind4, ind5)])
    ind_all = ind_all.reshape(_NT * _B, _NCH, 128)
    mask_i32 = mask.astype(jnp.int32).reshape(_B, _NCH, 128)
    partials = _ae_pull(*flats, ind_all, mask_i32)
    pull = jnp.sum(partials)
    push = jnp.zeros((), jnp.float32)
    return pull, push


# speedup vs baseline: 2.1824x; 1.7847x over previous
import functools

import jax
import jax.numpy as jnp
from jax import lax
from jax.experimental import pallas as pl
from jax.experimental.pallas import tpu as pltpu
from jax.experimental.pallas import tpu_sc as plsc

_B, _C, _H, _W, _N = 64, 1, 256, 256, 256
_HW = _H * _W
_NC, _NS, _L = 2, 16, 16
_NW = _NC * _NS
_RPW = _B // _NW
_NCH = _N // 128
_NT = 5


def _pull_body(t1, t2, t3, t4, t5, ind_hbm, mask_hbm, out_hbm,
               tile_v, idx_v, m_v, acc_v, sem):
    tags = (t1, t2, t3, t4, t5)
    wid = lax.axis_index("s") * _NC + lax.axis_index("c")

    acc = jnp.zeros((_L,), jnp.float32)
    for rb in range(_RPW):
        b = wid * _RPW + rb
        pltpu.sync_copy(mask_hbm.at[b], m_v)
        for t in range(_NT):
            pltpu.sync_copy(ind_hbm.at[t * _B + b], idx_v.at[t])

        num_vec = jnp.zeros((_L,), jnp.int32)
        for c in range(_NCH):
            for k in range(128 // _L):
                mb = m_v[c, pl.ds(k * _L, _L)] > 0
                num_vec = num_vec + plsc.all_reduce_population_count(mb)
        scale = 1.0 / (num_vec.astype(jnp.float32) + 1e-4)

        g_chunks = [[[] for _ in range(128 // _L)] for _ in range(_NCH)]
        for t in range(_NT):
            pltpu.sync_copy(tags[t].at[b], tile_v)
            for c in range(_NCH):
                for k in range(128 // _L):
                    sl = pl.ds(k * _L, _L)
                    ind = idx_v[t, c, sl]
                    ih = lax.shift_right_logical(ind, 8)
                    iw = lax.bitwise_and(ind, 255)
                    g_chunks[c][k].append(plsc.load_gather(tile_v, [ih, iw]))

        for c in range(_NCH):
            for k in range(128 // _L):
                g = g_chunks[c][k]
                tm = (g[0] + g[1] + g[2] + g[3] + g[4]) * 0.2
                ssd = jnp.zeros((_L,), jnp.float32)
                for t in range(_NT):
                    d = g[t] - tm
                    ssd = ssd + d * d
                mf = m_v[c, pl.ds(k * _L, _L)].astype(jnp.float32)
                acc = acc + (mf * ssd) * scale

    acc_v[...] = acc
    pltpu.sync_copy(acc_v, out_hbm.at[wid])


@jax.jit
def _ae_pull(t1, t2, t3, t4, t5, ind_all, mask_i32):
    mesh = plsc.VectorSubcoreMesh(core_axis_name="c", subcore_axis_name="s")
    run = functools.partial(
        pl.kernel,
        mesh=mesh,
        compiler_params=pltpu.CompilerParams(
            needs_layout_passes=False, use_tc_tiling_on_sc=True),
        out_type=jax.ShapeDtypeStruct((_NW, _L), jnp.float32),
        scratch_types=[
            pltpu.VMEM((_H, _W), jnp.float32),
            pltpu.VMEM((_NT, _NCH, 128), jnp.int32),
            pltpu.VMEM((_NCH, 128), jnp.int32),
            pltpu.VMEM((_L,), jnp.float32),
            pltpu.SemaphoreType.DMA,
        ],
    )(_pull_body)
    return run(t1, t2, t3, t4, t5, ind_all, mask_i32)


def kernel(tag1, tag2, tag3, tag4, tag5, ind1, ind2, ind3, ind4, ind5, mask):
    t3d = [t.reshape(_B, _H, _W) for t in (tag1, tag2, tag3, tag4, tag5)]
    ind_all = jnp.stack([ind1, ind2, ind3, ind4, ind5])
    ind_all = ind_all.reshape(_NT * _B, _NCH, 128)
    mask_i32 = mask.astype(jnp.int32).reshape(_B, _NCH, 128)
    partials = _ae_pull(*t3d, ind_all, mask_i32)
    pull = jnp.sum(partials)
    push = jnp.zeros((), jnp.float32)
    return pull, push


# speedup vs baseline: 2.5233x; 1.1562x over previous
import functools

import jax
import jax.numpy as jnp
from jax import lax
from jax.experimental import pallas as pl
from jax.experimental.pallas import tpu as pltpu
from jax.experimental.pallas import tpu_sc as plsc

_B, _C, _H, _W, _N = 64, 1, 256, 256, 256
_NC, _NS, _L = 2, 16, 16
_NW = _NC * _NS
_RPW = _B // _NW
_NCHUNK = _N // _L
_NT = 5


def _pull_body(t1, t2, t3, t4, t5, i1, i2, i3, i4, i5, mask_hbm, out_hbm,
               tile_v, idx_v, m_v, acc_v, sem, semi):
    tags = (t1, t2, t3, t4, t5)
    inds = (i1, i2, i3, i4, i5)
    wid = lax.axis_index("s") * _NC + lax.axis_index("c")

    acc = jnp.zeros((_L,), jnp.float32)
    for rb in range(_RPW):
        b = wid * _RPW + rb
        small = [pltpu.async_copy(mask_hbm.at[pl.ds(b, 1)], m_v, semi)]
        for t in range(_NT):
            small.append(pltpu.async_copy(
                inds[t].at[pl.ds(b, 1)], idx_v.at[t], semi))
        pltpu.async_copy(tags[0].at[b], tile_v, sem).wait()
        for cp in small:
            cp.wait()

        num_vec = jnp.zeros((_L,), jnp.int32)
        for k in range(_NCHUNK):
            mb = m_v[0, pl.ds(k * _L, _L)] > 0
            num_vec = num_vec + plsc.all_reduce_population_count(mb)
        scale = 1.0 / (num_vec.astype(jnp.float32) + 1e-4)

        s1 = [jnp.zeros((_L,), jnp.float32) for _ in range(_NCHUNK)]
        s2 = [jnp.zeros((_L,), jnp.float32) for _ in range(_NCHUNK)]
        for t in range(_NT):
            for k in range(_NCHUNK):
                ind = idx_v[t, 0, pl.ds(k * _L, _L)]
                ih = lax.shift_right_logical(ind, 8)
                iw = lax.bitwise_and(ind, 255)
                g = plsc.load_gather(tile_v, [ih, iw])
                s1[k] = s1[k] + g
                s2[k] = s2[k] + g * g
            if t + 1 < _NT:
                pltpu.sync_copy(tags[t + 1].at[b], tile_v)

        for k in range(_NCHUNK):
            ssd = s2[k] - s1[k] * s1[k] * 0.2
            mf = m_v[0, pl.ds(k * _L, _L)].astype(jnp.float32)
            acc = acc + (mf * ssd) * scale

    acc_v[...] = acc
    pltpu.sync_copy(acc_v, out_hbm.at[wid])


@jax.jit
def _ae_pull(t1, t2, t3, t4, t5, i1, i2, i3, i4, i5, mask_i32):
    mesh = plsc.VectorSubcoreMesh(core_axis_name="c", subcore_axis_name="s")
    run = functools.partial(
        pl.kernel,
        mesh=mesh,
        compiler_params=pltpu.CompilerParams(
            needs_layout_passes=False, use_tc_tiling_on_sc=True),
        out_type=jax.ShapeDtypeStruct((_NW, _L), jnp.float32),
        scratch_types=[
            pltpu.VMEM((_H, _W), jnp.float32),
            pltpu.VMEM((_NT, 1, _N), jnp.int32),
            pltpu.VMEM((1, _N), jnp.int32),
            pltpu.VMEM((_L,), jnp.float32),
            pltpu.SemaphoreType.DMA,
            pltpu.SemaphoreType.DMA,
        ],
    )(_pull_body)
    return run(t1, t2, t3, t4, t5, i1, i2, i3, i4, i5, mask_i32)


def kernel(tag1, tag2, tag3, tag4, tag5, ind1, ind2, ind3, ind4, ind5, mask):
    t3d = [t.reshape(_B, _H, _W) for t in (tag1, tag2, tag3, tag4, tag5)]
    mask_i32 = mask.astype(jnp.int32)
    partials = _ae_pull(*t3d, ind1, ind2, ind3, ind4, ind5, mask_i32)
    pull = jnp.sum(partials)
    push = jnp.zeros((), jnp.float32)
    return pull, push
